# baseline (speedup 1.0000x reference)
import functools

import jax
import jax.numpy as jnp
from jax import lax
from jax.experimental import pallas as pl
from jax.experimental.pallas import tpu as pltpu
from jax.experimental.pallas import tpu_sc as plsc

N = 10000
D = 128
E = 320000

NC = 2
NS = 16
NW = NC * NS
NPAD = 10240
K = 128
CPT = 80
EPG = K * CPT
EPAD = NW * EPG
CPS = 16
NSUP = (CPT + CPS - 1) // CPS
ZROWS = NPAD // NS
NSLOT = 2
BLK = 2048

_sc_mesh = plsc.VectorSubcoreMesh(core_axis_name="c", subcore_axis_name="s")


def _segsum_body(x_hbm, src_hbm, dst_hbm, zeros_hbm, out_hbm,
                 agg_sh, srcb0, srcb1, dstb0, dstb1, rows,
                 isem0, isem1, gsem, ssem):
    cid = lax.axis_index("c")
    sid = lax.axis_index("s")
    wid = cid * NS + sid
    chunk0 = wid * CPT

    pltpu.sync_copy(zeros_hbm, agg_sh.at[pl.ds(sid * ZROWS, ZROWS)])
    plsc.subcore_barrier()

    srcbs = (srcb0, srcb1)
    dstbs = (dstb0, dstb1)
    isems = (isem0, isem1)

    def idx_copies(s, sl):
        start = chunk0 + s * CPS
        ncp = min(CPS, CPT - s * CPS)
        return (pltpu.make_async_copy(src_hbm.at[pl.ds(start, ncp)],
                                      srcbs[sl].at[pl.ds(0, ncp)], isems[sl]),
                pltpu.make_async_copy(dst_hbm.at[pl.ds(start, ncp)],
                                      dstbs[sl].at[pl.ds(0, ncp)], isems[sl]))

    def gather(sl, kk, gl):
        return pltpu.make_async_copy(
            x_hbm.at[srcbs[sl].at[kk]], rows.at[gl], gsem.at[gl])

    def scat_start(sl, kk, gl):
        pltpu.async_copy(rows.at[gl], agg_sh.at[dstbs[sl].at[kk]],
                         ssem.at[gl], add=True)

    def scat_wait(sl, kk, gl):
        pltpu.make_async_copy(rows.at[gl], agg_sh.at[dstbs[sl].at[kk]],
                              ssem.at[gl]).wait()

    for cp in idx_copies(0, 0):
        cp.start()

    for s in range(NSUP):
        sl = s % 2
        ng = min(CPS, CPT - s * CPS)
        for cp in idx_copies(s, sl):
            cp.wait()
        if s + 1 < NSUP:
            for cp in idx_copies(s + 1, 1 - sl):
                cp.start()

        for gl in range(min(NSLOT, ng)):
            gather(sl, gl, gl).start()

        for kk in range(ng):
            gl = kk % NSLOT
            gather(sl, kk, gl).wait()
            scat_start(sl, kk, gl)
            scat_wait(sl, kk, gl)
            if kk + NSLOT < ng:
                gather(sl, kk + NSLOT, gl).start()

    plsc.subcore_barrier()
    pltpu.sync_copy(agg_sh.at[pl.ds(sid * ZROWS, ZROWS)],
                    out_hbm.at[cid].at[pl.ds(sid * ZROWS, ZROWS)])


@jax.jit
def _segsum_sc(xp, src_p, dst_p, zeros):
    return pl.kernel(
        _segsum_body,
        out_type=jax.ShapeDtypeStruct((NC, NPAD, D), jnp.float32),
        mesh=_sc_mesh,
        scratch_types=[
            pltpu.VMEM_SHARED((NPAD, D), jnp.float32),
            pltpu.VMEM((CPS, K), jnp.int32),
            pltpu.VMEM((CPS, K), jnp.int32),
            pltpu.VMEM((CPS, K), jnp.int32),
            pltpu.VMEM((CPS, K), jnp.int32),
            pltpu.VMEM((NSLOT, K, D), jnp.float32),
            pltpu.SemaphoreType.DMA,
            pltpu.SemaphoreType.DMA,
            pltpu.SemaphoreType.DMA((NSLOT,)),
            pltpu.SemaphoreType.DMA((NSLOT,)),
        ],
    )(xp, src_p, dst_p, zeros)


def _dense_body(x_ref, agg_ref, ws_ref, wn_ref, b_ref, o_ref, *, mode):
    acc = jnp.dot(x_ref[...], ws_ref[...], preferred_element_type=jnp.float32)
    agg = agg_ref[0] + agg_ref[1]
    acc += jnp.dot(agg, wn_ref[...], preferred_element_type=jnp.float32)
    acc += b_ref[...]
    if mode == "relu":
        o_ref[...] = jnp.maximum(acc, 0.0)
    else:
        o_ref[...] = acc


def _dense_res_body(x_ref, agg_ref, ws_ref, wn_ref, b_ref, temp_ref, o_ref):
    acc = jnp.dot(x_ref[...], ws_ref[...], preferred_element_type=jnp.float32)
    agg = agg_ref[0] + agg_ref[1]
    acc += jnp.dot(agg, wn_ref[...], preferred_element_type=jnp.float32)
    acc += b_ref[...]
    o_ref[...] = (temp_ref[...] + jnp.maximum(acc, 0.0)) * 0.5


def _row_spec():
    return pl.BlockSpec((BLK, D), lambda i: (i, 0))


def _agg_spec():
    return pl.BlockSpec((NC, BLK, D), lambda i: (0, i, 0))


def _w_spec():
    return pl.BlockSpec((D, D), lambda i: (0, 0))


def _b_spec():
    return pl.BlockSpec((1, D), lambda i: (0, 0))


@functools.partial(jax.jit, static_argnames=("mode",))
def _dense_layer(x, agg, ws, wn, b, mode):
    return pl.pallas_call(
        functools.partial(_dense_body, mode=mode),
        grid=(NPAD // BLK,),
        in_specs=[_row_spec(), _agg_spec(), _w_spec(), _w_spec(), _b_spec()],
        out_specs=_row_spec(),
        out_shape=jax.ShapeDtypeStruct((NPAD, D), jnp.float32),
    )(x, agg, ws, wn, b.reshape(1, D))


@jax.jit
def _dense_res_layer(x, agg, ws, wn, b, temp):
    return pl.pallas_call(
        _dense_res_body,
        grid=(NPAD // BLK,),
        in_specs=[_row_spec(), _agg_spec(), _w_spec(), _w_spec(), _b_spec(),
                  _row_spec()],
        out_specs=_row_spec(),
        out_shape=jax.ShapeDtypeStruct((NPAD, D), jnp.float32),
    )(x, agg, ws, wn, b.reshape(1, D), temp)


def kernel(edges, shape_features, Ws_self, Ws_neigh, bs, Wf_self, Wf_neigh, bf):
    src = edges[0]
    dst = edges[1]

    src_p = jnp.concatenate(
        [src, jnp.zeros((EPAD - E,), jnp.int32)]).reshape(EPAD // K, K)
    dst_p = jnp.concatenate(
        [dst, jnp.full((EPAD - E,), N, jnp.int32)]).reshape(EPAD // K, K)
    zeros = jnp.zeros((ZROWS, D), jnp.float32)

    xp = jnp.zeros((NPAD, D), jnp.float32).at[:N].set(shape_features)

    x = _dense_layer(xp, _segsum_sc(xp, src_p, dst_p, zeros),
                     Ws_self[0], Ws_neigh[0], bs[0], mode="relu")
    for i in range(1, 12, 2):
        temp = x
        x = _dense_layer(x, _segsum_sc(x, src_p, dst_p, zeros),
                         Ws_self[i], Ws_neigh[i], bs[i], mode="relu")
        x = _dense_res_layer(x, _segsum_sc(x, src_p, dst_p, zeros),
                             Ws_self[i + 1], Ws_neigh[i + 1], bs[i + 1], temp)

    wf_s = jnp.zeros((D, D), jnp.float32).at[:, :3].set(Wf_self)
    wf_n = jnp.zeros((D, D), jnp.float32).at[:, :3].set(Wf_neigh)
    bf_p = jnp.zeros((D,), jnp.float32).at[:3].set(bf)
    coords_p = _dense_layer(x, _segsum_sc(x, src_p, dst_p, zeros),
                            wf_s, wf_n, bf_p, mode="linear")
    return (x[:N], coords_p[:N, :3])
